# baseline (speedup 1.0000x reference)
import jax
import jax.numpy as jnp
import numpy as np
from jax import lax
from jax.experimental import pallas as pl
from jax.experimental.pallas import tpu as pltpu

_ROWS = 8
_THREEFRY_C = 0x1BD11BDA
_F32_ONE_BITS = 0x3F800000
_TINY = np.float32(np.finfo(np.float32).tiny)


def _rotl(x, r):
    return (x << jnp.int32(r)) | lax.shift_right_logical(x, jnp.int32(32 - r))


def _threefry2x32(k0, k1, c1):
    k2 = k0 ^ k1 ^ jnp.int32(_THREEFRY_C)
    x0 = k0
    x1 = c1 + k1
    ks = (k0, k1, k2)
    rots = ((13, 15, 26, 6), (17, 29, 16, 24),
            (13, 15, 26, 6), (17, 29, 16, 24), (13, 15, 26, 6))
    for d in range(5):
        for r in rots[d]:
            x0 = x0 + x1
            x1 = _rotl(x1, r) ^ x0
        x0 = x0 + ks[(d + 1) % 3]
        x1 = x1 + ks[(d + 2) % 3] + jnp.int32(d + 1)
    return x0 ^ x1


def _bits_to_unit_float(bits):
    fb = lax.shift_right_logical(bits, jnp.int32(9)) | jnp.int32(_F32_ONE_BITS)
    return lax.bitcast_convert_type(fb, jnp.float32) - jnp.float32(1.0)


def _body(scal_ref, logits_ref, target_ref, out_ref):
    k = pl.program_id(0)
    key0 = scal_ref[0]
    key1 = scal_ref[1]
    ckey0 = scal_ref[2]
    ckey1 = scal_ref[3]
    step = scal_ref[4]
    thr = lax.bitcast_convert_type(scal_ref[5], jnp.float32)

    x = logits_ref[...]
    R, V = x.shape

    row = lax.broadcasted_iota(jnp.int32, (R, V), 0)
    col = lax.broadcasted_iota(jnp.int32, (R, V), 1)
    ctr = (k * R) * V + row * V + col
    bits = _threefry2x32(key0, key1, ctr)
    floats = _bits_to_unit_float(bits)
    u = jnp.maximum(_TINY, floats * (jnp.float32(1.0) - _TINY) + _TINY)
    g = -jnp.log(-jnp.log(u))

    m = jnp.max(x, axis=1, keepdims=True)
    shifted = x - m
    lse = jnp.log(jnp.sum(jnp.exp(shifted), axis=1, keepdims=True))
    score = (shifted - lse) + g

    best = jnp.max(score, axis=1, keepdims=True)
    idx = jnp.min(jnp.where(score == best, col, jnp.int32(2**31 - 1)),
                  axis=1, keepdims=True)
    sample = idx.astype(jnp.float32)

    rctr = lax.broadcasted_iota(jnp.int32, (R, 1), 0) + k * R
    cbits = _threefry2x32(ckey0, ckey1, rctr)
    cp = jnp.maximum(jnp.float32(0.0), _bits_to_unit_float(cbits))

    t = target_ref[...]
    tcol = lax.broadcasted_iota(jnp.int32, t.shape, 1)
    tgt = jnp.sum(jnp.where(tcol == step, t, jnp.float32(0.0)),
                  axis=1, keepdims=True)

    out_ref[...] = jnp.where(cp < thr, tgt, sample)


def kernel(target, logits, step, summary_step):
    B, V = logits.shape
    T = target.shape[1]

    skd = lax.bitcast_convert_type(
        jax.random.key_data(jax.random.fold_in(jax.random.key(42), summary_step)),
        jnp.int32)
    ckd = lax.bitcast_convert_type(
        jax.random.key_data(jax.random.fold_in(jax.random.key(7), step)),
        jnp.int32)
    stepf = jnp.asarray(step, jnp.float32)
    thr = jnp.float32(100.0) / (jnp.float32(100.0) + jnp.exp(stepf / jnp.float32(100.0)))
    scalars = jnp.concatenate([
        skd.reshape(2), ckd.reshape(2),
        jnp.asarray(step, jnp.int32).reshape(1),
        lax.bitcast_convert_type(thr, jnp.int32).reshape(1),
    ])

    grid = (B // _ROWS,)
    out = pl.pallas_call(
        _body,
        grid=grid,
        in_specs=[
            pl.BlockSpec(memory_space=pltpu.SMEM),
            pl.BlockSpec((_ROWS, V), lambda k: (k, 0)),
            pl.BlockSpec((_ROWS, T), lambda k: (k, 0)),
        ],
        out_specs=pl.BlockSpec((_ROWS, 1), lambda k: (k, 0)),
        out_shape=jax.ShapeDtypeStruct((B, 1), jnp.float32),
    )(scalars, logits, target)
    return out


# speedup vs baseline: 1.0444x; 1.0444x over previous
import jax
import jax.numpy as jnp
import numpy as np
from jax import lax
from jax.experimental import pallas as pl
from jax.experimental.pallas import tpu as pltpu

_ROWS = 8
_THREEFRY_C = 0x1BD11BDA
_F32_ONE_BITS = 0x3F800000
_TINY = np.float32(np.finfo(np.float32).tiny)


def _rotl(x, r):
    return (x << jnp.int32(r)) | lax.shift_right_logical(x, jnp.int32(32 - r))


def _threefry2x32(k0, k1, c1):
    k2 = k0 ^ k1 ^ jnp.int32(_THREEFRY_C)
    x0 = k0
    x1 = c1 + k1
    ks = (k0, k1, k2)
    rots = ((13, 15, 26, 6), (17, 29, 16, 24),
            (13, 15, 26, 6), (17, 29, 16, 24), (13, 15, 26, 6))
    for d in range(5):
        for r in rots[d]:
            x0 = x0 + x1
            x1 = _rotl(x1, r) ^ x0
        x0 = x0 + ks[(d + 1) % 3]
        x1 = x1 + ks[(d + 2) % 3] + jnp.int32(d + 1)
    return x0 ^ x1


def _bits_to_unit_float(bits):
    fb = lax.shift_right_logical(bits, jnp.int32(9)) | jnp.int32(_F32_ONE_BITS)
    return lax.bitcast_convert_type(fb, jnp.float32) - jnp.float32(1.0)


def _body(scal_ref, logits_ref, target_ref, out_ref):
    k = pl.program_id(0)
    key0 = scal_ref[0]
    key1 = scal_ref[1]
    ckey0 = scal_ref[2]
    ckey1 = scal_ref[3]
    step = scal_ref[4]
    thr = lax.bitcast_convert_type(scal_ref[5], jnp.float32)

    x = logits_ref[...]
    R, V = x.shape

    rowoff = (lax.broadcasted_iota(jnp.int32, (R, 1), 0) + k * R) * V
    col = lax.broadcasted_iota(jnp.int32, (R, V), 1)
    ctr = rowoff + col
    bits = _threefry2x32(key0, key1, ctr)
    floats = _bits_to_unit_float(bits)
    u = jnp.maximum(_TINY, floats * (jnp.float32(1.0) - _TINY) + _TINY)
    g = -jnp.log(-jnp.log(u))

    score = x + g

    best = jnp.max(score, axis=1, keepdims=True)
    idx = jnp.min(jnp.where(score == best, col, jnp.int32(2**31 - 1)),
                  axis=1, keepdims=True)
    sample = idx.astype(jnp.float32)

    rctr = lax.broadcasted_iota(jnp.int32, (R, 1), 0) + k * R
    cbits = _threefry2x32(ckey0, ckey1, rctr)
    cp = jnp.maximum(jnp.float32(0.0), _bits_to_unit_float(cbits))

    t = target_ref[...]
    tcol = lax.broadcasted_iota(jnp.int32, t.shape, 1)
    tgt = jnp.sum(jnp.where(tcol == step, t, jnp.float32(0.0)),
                  axis=1, keepdims=True)

    out_ref[...] = jnp.where(cp < thr, tgt, sample)


def kernel(target, logits, step, summary_step):
    B, V = logits.shape
    T = target.shape[1]

    skd = lax.bitcast_convert_type(
        jax.random.key_data(jax.random.fold_in(jax.random.key(42), summary_step)),
        jnp.int32)
    ckd = lax.bitcast_convert_type(
        jax.random.key_data(jax.random.fold_in(jax.random.key(7), step)),
        jnp.int32)
    stepf = jnp.asarray(step, jnp.float32)
    thr = jnp.float32(100.0) / (jnp.float32(100.0) + jnp.exp(stepf / jnp.float32(100.0)))
    scalars = jnp.concatenate([
        skd.reshape(2), ckd.reshape(2),
        jnp.asarray(step, jnp.int32).reshape(1),
        lax.bitcast_convert_type(thr, jnp.int32).reshape(1),
    ])

    grid = (B // _ROWS,)
    out = pl.pallas_call(
        _body,
        grid=grid,
        in_specs=[
            pl.BlockSpec(memory_space=pltpu.SMEM),
            pl.BlockSpec((_ROWS, V), lambda k: (k, 0)),
            pl.BlockSpec((_ROWS, T), lambda k: (k, 0)),
        ],
        out_specs=pl.BlockSpec((_ROWS, 1), lambda k: (k, 0)),
        out_shape=jax.ShapeDtypeStruct((B, 1), jnp.float32),
    )(scalars, logits, target)
    return out
